# Initial kernel scaffold; baseline (speedup 1.0000x reference)
import jax
import jax.numpy as jnp
from jax.experimental import pallas as pl


def kernel(x, edge_index, gnn_batch, W1, b1, W2, b2, W3, b3, W4, b4, conv5_w, conv5_b, conv6_w, conv6_b, cls1_W, cls1_b, cls2_W, cls2_b):
    raise NotImplementedError("write your pallas kernel here")


# speedup vs baseline: 17.6126x; 17.6126x over previous
import functools

import jax
import jax.numpy as jnp
from jax import lax
from jax.experimental import pallas as pl
from jax.experimental.pallas import tpu as pltpu
from jax.experimental.pallas import tpu_sc as plsc

N = 10000
E = 320000
D = 128
SORTK = 30
NUM_GRAPHS = 11

NC = 2
NS = 16
NW = NC * NS
CB = 64
CH = 160
BI = 16
EW = CH * CB
E_PAD = NW * EW
N_ACC = 10240
ROWS_PER_TILE = N_ACC // NS
NCOL = 10112
INF = 3e38


def _sc_edge_pass_wide(src3, dst3, g, zero_init):
  mesh = plsc.VectorSubcoreMesh(core_axis_name="c", subcore_axis_name="s", num_cores=NC, num_subcores=NS)

  @functools.partial(
      pl.kernel,
      out_type=jax.ShapeDtypeStruct((NC, N_ACC, D), jnp.float32),
      mesh=mesh,
      scratch_types=[
          pltpu.VMEM((BI, CB), jnp.int32),
          pltpu.VMEM((BI, CB), jnp.int32),
          pltpu.VMEM((CB, D), jnp.float32),
          pltpu.VMEM((CB, D), jnp.float32),
          pltpu.VMEM_SHARED((N_ACC, D), jnp.float32),
          pltpu.SemaphoreType.DMA,
          pltpu.SemaphoreType.DMA,
      ],
  )
  def k(src_hbm, dst_hbm, g_hbm, zero_hbm, out_hbm, srcv, dstv, bufa, bufb,
        acc, sga, sgb):
    cid = lax.axis_index("c")
    sid = lax.axis_index("s")
    wid = sid * NC + cid
    row0 = sid * ROWS_PER_TILE
    pltpu.sync_copy(zero_hbm.at[pl.ds(row0, ROWS_PER_TILE)],
                    acc.at[pl.ds(row0, ROWS_PER_TILE)])
    plsc.subcore_barrier()

    def blk(b, carry):
      pltpu.sync_copy(src_hbm.at[wid, pl.ds(b * BI, BI)], srcv)
      pltpu.sync_copy(dst_hbm.at[wid, pl.ds(b * BI, BI)], dstv)

      def body(i, c2):
        j = i * 2
        da = pltpu.async_copy(g_hbm.at[srcv.at[j]], bufa, sga)
        db = pltpu.async_copy(g_hbm.at[srcv.at[j + 1]], bufb, sgb)
        da.wait()
        pltpu.sync_copy(bufa, acc.at[dstv.at[j]], add=True)
        db.wait()
        pltpu.sync_copy(bufb, acc.at[dstv.at[j + 1]], add=True)
        return c2

      lax.fori_loop(0, BI // 2, body, 0)
      return carry

    lax.fori_loop(0, CH // BI, blk, 0)
    plsc.subcore_barrier()
    pltpu.sync_copy(acc.at[pl.ds(row0, ROWS_PER_TILE)],
                    out_hbm.at[cid, pl.ds(row0, ROWS_PER_TILE)])

  return k(src3, dst3, g, zero_init)


def _sc_edge_pass_scalar(src3, dst3, g, zero_init):
  mesh = plsc.VectorSubcoreMesh(core_axis_name="c", subcore_axis_name="s", num_cores=NC, num_subcores=NS)

  @functools.partial(
      pl.kernel,
      out_type=jax.ShapeDtypeStruct((NC, N_ACC), jnp.float32),
      mesh=mesh,
      scratch_types=[
          pltpu.VMEM((CH, CB), jnp.int32),
          pltpu.VMEM((CH, CB), jnp.int32),
          pltpu.VMEM((CB,), jnp.float32),
          pltpu.VMEM((CB,), jnp.float32),
          pltpu.VMEM_SHARED((N_ACC,), jnp.float32),
          pltpu.SemaphoreType.DMA,
          pltpu.SemaphoreType.DMA,
      ],
  )
  def k(src_hbm, dst_hbm, g_hbm, zero_hbm, out_hbm, srcv, dstv, bufa, bufb,
        acc, sga, sgb):
    cid = lax.axis_index("c")
    sid = lax.axis_index("s")
    wid = sid * NC + cid
    pltpu.sync_copy(src_hbm.at[wid], srcv)
    pltpu.sync_copy(dst_hbm.at[wid], dstv)
    row0 = sid * ROWS_PER_TILE
    pltpu.sync_copy(zero_hbm.at[pl.ds(row0, ROWS_PER_TILE)],
                    acc.at[pl.ds(row0, ROWS_PER_TILE)])
    plsc.subcore_barrier()

    def body(i, carry):
      j = i * 2
      da = pltpu.async_copy(g_hbm.at[srcv.at[j]], bufa, sga)
      db = pltpu.async_copy(g_hbm.at[srcv.at[j + 1]], bufb, sgb)
      da.wait()
      pltpu.sync_copy(bufa, acc.at[dstv.at[j]], add=True)
      db.wait()
      pltpu.sync_copy(bufb, acc.at[dstv.at[j + 1]], add=True)
      return carry

    lax.fori_loop(0, CH // 2, body, 0)
    plsc.subcore_barrier()
    pltpu.sync_copy(acc.at[pl.ds(row0, ROWS_PER_TILE)],
                    out_hbm.at[cid, pl.ds(row0, ROWS_PER_TILE)])

  return k(src3, dst3, g, zero_init)


def _sc_degree(dst3, zero_init):
  mesh = plsc.VectorSubcoreMesh(core_axis_name="c", subcore_axis_name="s", num_cores=NC, num_subcores=NS)

  @functools.partial(
      pl.kernel,
      out_type=jax.ShapeDtypeStruct((NC, N_ACC), jnp.float32),
      mesh=mesh,
      scratch_types=[
          pltpu.VMEM((CH, CB), jnp.int32),
          pltpu.VMEM((CB,), jnp.float32),
          pltpu.VMEM_SHARED((N_ACC,), jnp.float32),
      ],
  )
  def k(dst_hbm, zero_hbm, out_hbm, dstv, ones_v, acc):
    cid = lax.axis_index("c")
    sid = lax.axis_index("s")
    wid = sid * NC + cid
    pltpu.sync_copy(dst_hbm.at[wid], dstv)
    for i in range(CB // 16):
      ones_v[pl.ds(i * 16, 16)] = jnp.ones((16,), jnp.float32)
    row0 = sid * ROWS_PER_TILE
    pltpu.sync_copy(zero_hbm.at[pl.ds(row0, ROWS_PER_TILE)],
                    acc.at[pl.ds(row0, ROWS_PER_TILE)])
    plsc.subcore_barrier()

    def body(j, carry):
      pltpu.sync_copy(ones_v, acc.at[dstv.at[j]], add=True)
      return carry

    lax.fori_loop(0, CH, body, 0)
    plsc.subcore_barrier()
    pltpu.sync_copy(acc.at[pl.ds(row0, ROWS_PER_TILE)],
                    out_hbm.at[cid, pl.ds(row0, ROWS_PER_TILE)])

  return k(dst3, zero_init)


def _tc_prep(degp, xp, W1):
  def k(degp_ref, x_ref, w_ref, dinv_ref, g_ref):
    deg = degp_ref[0] + degp_ref[1] + 1.0
    dinv = lax.rsqrt(deg)
    dinv_ref[...] = dinv
    h = jnp.dot(x_ref[...], w_ref[...], preferred_element_type=jnp.float32)
    g_ref[...] = h * dinv

  return pl.pallas_call(
      k,
      out_shape=(jax.ShapeDtypeStruct((N_ACC, 1), jnp.float32),
                 jax.ShapeDtypeStruct((N_ACC, D), jnp.float32)),
  )(degp, xp, W1)


def _tc_layer(partials, g_prev, dinv, b_prev, W_next):
  dn = W_next.shape[1]

  def k(p_ref, g_ref, dinv_ref, b_ref, w_ref, x_ref, gn_ref):
    dinv = dinv_ref[...]
    xcur = jnp.tanh(dinv * (p_ref[0] + p_ref[1] + g_ref[...]) + b_ref[...])
    x_ref[...] = xcur
    h = jnp.dot(xcur, w_ref[...], preferred_element_type=jnp.float32)
    gn_ref[...] = h * dinv

  return pl.pallas_call(
      k,
      out_shape=(jax.ShapeDtypeStruct((N_ACC, D), jnp.float32),
                 jax.ShapeDtypeStruct((N_ACC, dn), jnp.float32)),
  )(partials, g_prev, dinv, b_prev, W_next)


def _tc_pool(p4, g4, dinv, b4, x1, x2, x3, batch_pad):
  def k(p4_ref, g4_ref, dinv_ref, b4_ref, x1_ref, x2_ref, x3_ref, batch_ref,
        o1_ref, o2_ref, o3_ref, o4_ref, K_ref, S_ref):
    x4 = jnp.tanh(dinv_ref[...] * (p4_ref[0] + p4_ref[1] + g4_ref[...])
                  + b4_ref[...])
    x4row = x4[0:NCOL, 0][None, :]

    rowid = lax.broadcasted_iota(jnp.int32, (16, NCOL), 0)
    col = lax.broadcasted_iota(jnp.int32, (16, NCOL), 1)
    in_graph = batch_ref[...] == rowid
    key = rowid.astype(jnp.float32) * jnp.float32(1e6) - x4row
    K_ref[...] = jnp.where(in_graph, key, INF)
    counts = jnp.sum(in_graph.astype(jnp.int32), axis=1, keepdims=True)
    S_ref[pl.ds(NUM_GRAPHS * SORTK, 6), :] = jnp.zeros((6, NCOL), jnp.float32)

    def step(t, carry):
      K = K_ref[...]
      m = jnp.min(K, axis=1, keepdims=True)
      hit = K == m
      a = jnp.min(jnp.where(hit, col, NCOL), axis=1, keepdims=True)
      sel = (col == a).astype(jnp.float32)
      valid = (t < counts).astype(jnp.float32)
      rows = sel * valid
      for g in range(NUM_GRAPHS):
        S_ref[pl.ds(g * SORTK + t, 1), :] = rows[g][None, :]
      K_ref[...] = jnp.where(col == a, INF, K)
      return carry

    lax.fori_loop(0, SORTK, step, 0)

    S = S_ref[...]
    o1_ref[...] = jnp.dot(S, x1_ref[0:NCOL, :], preferred_element_type=jnp.float32)
    o2_ref[...] = jnp.dot(S, x2_ref[0:NCOL, :], preferred_element_type=jnp.float32)
    o3_ref[...] = jnp.dot(S, x3_ref[0:NCOL, :], preferred_element_type=jnp.float32)
    o4_ref[...] = jnp.dot(S, x4[0:NCOL, :], preferred_element_type=jnp.float32)

  return pl.pallas_call(
      k,
      out_shape=(jax.ShapeDtypeStruct((336, D), jnp.float32),
                 jax.ShapeDtypeStruct((336, D), jnp.float32),
                 jax.ShapeDtypeStruct((336, D), jnp.float32),
                 jax.ShapeDtypeStruct((336, 1), jnp.float32)),
      scratch_shapes=[
          pltpu.VMEM((16, NCOL), jnp.float32),
          pltpu.VMEM((336, NCOL), jnp.float32),
      ],
  )(p4, g4, dinv, b4, x1, x2, x3, batch_pad)


def _tc_dense(P1, P2, P3, p4p, W5a, W5b, W5c, w5d, b5, W6r, b6, C3, cb1,
              cls2_W, cb2):
  def k(p1_ref, p2_ref, p3_ref, p4p_ref, w5a_ref, w5b_ref, w5c_ref, w5d_ref,
        b5_ref, w6_ref, b6_ref, c3_ref, cb1_ref, w2c_ref, cb2_ref, out_ref):
    P1, P2, P3, p4p = p1_ref[...], p2_ref[...], p3_ref[...], p4p_ref[...]
    o5 = (jnp.dot(P1, w5a_ref[...], preferred_element_type=jnp.float32)
          + jnp.dot(P2, w5b_ref[...], preferred_element_type=jnp.float32)
          + jnp.dot(P3, w5c_ref[...], preferred_element_type=jnp.float32)
          + p4p * w5d_ref[...] + b5_ref[...])
    o5 = jnp.maximum(o5, 0.0)

    ei = lax.broadcasted_iota(jnp.int32, (165, 336), 0)
    ej = lax.broadcasted_iota(jnp.int32, (165, 336), 1)
    gidx = ei // 15
    pidx = ei - gidx * 15
    sel_e = (ej == gidx * SORTK + 2 * pidx).astype(jnp.float32)
    sel_o = (ej == gidx * SORTK + 2 * pidx + 1).astype(jnp.float32)
    Me = jnp.dot(sel_e, o5, preferred_element_type=jnp.float32)
    Mo = jnp.dot(sel_o, o5, preferred_element_type=jnp.float32)
    M = jnp.maximum(Me, Mo)

    wi = lax.broadcasted_iota(jnp.int32, (121, 165), 0)
    wj = lax.broadcasted_iota(jnp.int32, (121, 165), 1)
    wg = wi // 11
    wp = wi - wg * 11
    wins = []
    for j in range(5):
      selj = (wj == wg * 15 + wp + j).astype(jnp.float32)
      wins.append(jnp.dot(selj, M, preferred_element_type=jnp.float32))
    win = jnp.concatenate(wins, axis=1)
    z = jnp.dot(win, w6_ref[...], preferred_element_type=jnp.float32)
    z = jnp.maximum(z + b6_ref[...], 0.0)

    h = jnp.sum(z[:, :, None] * c3_ref[...], axis=(0, 1), keepdims=True)
    h = jnp.maximum(h[0] + cb1_ref[...], 0.0)
    logits = jnp.dot(h, w2c_ref[...],
                     preferred_element_type=jnp.float32) + cb2_ref[...]
    mx = jnp.max(logits, axis=1, keepdims=True)
    ex = jnp.exp(logits - mx)
    out_ref[...] = ex / jnp.sum(ex, axis=1, keepdims=True)

  return pl.pallas_call(
      k,
      out_shape=jax.ShapeDtypeStruct((1, 10), jnp.float32),
  )(P1, P2, P3, p4p, W5a, W5b, W5c, w5d, b5, W6r, b6, C3, cb1, cls2_W, cb2)


def kernel(x, edge_index, gnn_batch, W1, b1, W2, b2, W3, b3, W4, b4,
           conv5_w, conv5_b, conv6_w, conv6_b, cls1_W, cls1_b, cls2_W, cls2_b):
  f32 = jnp.float32
  src, dst = edge_index[0], edge_index[1]
  pad_idx = (N + (jnp.arange(E_PAD - E, dtype=jnp.int32) % 128))
  src3 = jnp.concatenate([src, pad_idx]).reshape(NW, CH, CB)
  dst3 = jnp.concatenate([dst, pad_idx]).reshape(NW, CH, CB)

  xp = jnp.zeros((N_ACC, x.shape[1]), f32).at[:N].set(x)
  zeros_w = jnp.zeros((N_ACC, D), f32)
  zeros_s = jnp.zeros((N_ACC,), f32)
  batch_pad = jnp.full((1, NCOL), 15, jnp.int32).at[0, :N].set(gnn_batch)

  W5r = conv5_w.reshape(64, 3 * D + 1)
  W5a, W5b, W5c = W5r[:, 0:128].T, W5r[:, 128:256].T, W5r[:, 256:384].T
  w5d = W5r[:, 384][None, :]
  W6r = conv6_w.transpose(2, 1, 0).reshape(320, 128)
  C3 = cls1_W.reshape(11, 128, 11, 128).transpose(0, 2, 1, 3).reshape(121, 128, 128)

  degp = _sc_degree(dst3, zeros_s)
  dinv, g1 = _tc_prep(degp.reshape(NC, N_ACC, 1), xp, W1)

  p1 = _sc_edge_pass_wide(src3, dst3, g1, zeros_w)
  x1, g2 = _tc_layer(p1, g1, dinv, b1[None, :], W2)
  p2 = _sc_edge_pass_wide(src3, dst3, g2, zeros_w)
  x2, g3 = _tc_layer(p2, g2, dinv, b2[None, :], W3)
  p3 = _sc_edge_pass_wide(src3, dst3, g3, zeros_w)
  x3, g4 = _tc_layer(p3, g3, dinv, b3[None, :], W4)

  p4 = _sc_edge_pass_scalar(src3, dst3, g4[:, 0], zeros_s)

  P1, P2, P3, p4p = _tc_pool(p4.reshape(NC, N_ACC, 1), g4, dinv, b4[None, :],
                             x1, x2, x3, batch_pad)
  probs = _tc_dense(P1, P2, P3, p4p, W5a, W5b, W5c, w5d, conv5_b[None, :],
                    W6r, conv6_b[None, :], C3, cls1_b[None, :], cls2_W,
                    cls2_b[None, :])
  return probs.reshape(10)


# speedup vs baseline: 21.7932x; 1.2374x over previous
import functools

import jax
import jax.numpy as jnp
from jax import lax
from jax.experimental import pallas as pl
from jax.experimental.pallas import tpu as pltpu
from jax.experimental.pallas import tpu_sc as plsc

N = 10000
E = 320000
D = 128
SORTK = 30
NUM_GRAPHS = 11

NC = 2
NS = 16
NW = NC * NS
CB = 80
CH = 128
BI = 8
EW = CH * CB
E_PAD = NW * EW
N_ACC = 10240
ROWS_PER_TILE = N_ACC // NS
NCOL = 10112
INF = 3e38


def _sc_edge_pass(src3, dst3, g, zero_init, wide):
  mesh = plsc.VectorSubcoreMesh(core_axis_name="c", subcore_axis_name="s",
                                num_cores=NC, num_subcores=NS)
  vshape = (N_ACC, D) if wide else (N_ACC,)
  bshape = (CB, D) if wide else (CB,)
  NBUF = 3

  @functools.partial(
      pl.kernel,
      out_type=jax.ShapeDtypeStruct((NC,) + vshape, jnp.float32),
      mesh=mesh,
      scratch_types=[
          pltpu.VMEM((BI, CB), jnp.int32),
          pltpu.VMEM((BI, CB), jnp.int32),
          pltpu.VMEM((NBUF,) + bshape, jnp.float32),
          pltpu.VMEM_SHARED(vshape, jnp.float32),
      ] + [pltpu.SemaphoreType.DMA] * (2 * NBUF),
  )
  def k(src_hbm, dst_hbm, g_hbm, zero_hbm, out_hbm, srcv, dstv, bufs,
        acc, *sems):
    gsems, ssems = sems[:NBUF], sems[NBUF:]
    cid = lax.axis_index("c")
    sid = lax.axis_index("s")
    wid = sid * NC + cid
    row0 = sid * ROWS_PER_TILE
    pltpu.sync_copy(zero_hbm.at[pl.ds(row0, ROWS_PER_TILE)],
                    acc.at[pl.ds(row0, ROWS_PER_TILE)])
    plsc.subcore_barrier()

    def blk(b, carry):
      pltpu.sync_copy(src_hbm.at[wid, pl.ds(b * BI, BI)], srcv)
      pltpu.sync_copy(dst_hbm.at[wid, pl.ds(b * BI, BI)], dstv)

      g_descs = [None] * BI
      s_descs = [None] * BI
      s_waited = [False] * BI
      L = NBUF - 1

      def gather(c):
        return pltpu.async_copy(g_hbm.at[srcv.at[c]], bufs.at[c % NBUF],
                                gsems[c % NBUF])

      for c in range(L):
        g_descs[c] = gather(c)
      for c in range(BI):
        p = c % NBUF
        g_descs[c].wait()
        s_descs[c] = pltpu.async_copy(bufs.at[p], acc.at[dstv.at[c]],
                                      ssems[p], add=True)
        n = c + L
        if n < BI:
          prev = n - NBUF
          if prev >= 0:
            s_descs[prev].wait()
            s_waited[prev] = True
          g_descs[n] = gather(n)
      for c in range(BI):
        if not s_waited[c]:
          s_descs[c].wait()
      return carry

    lax.fori_loop(0, CH // BI, blk, 0)
    plsc.subcore_barrier()
    pltpu.sync_copy(acc.at[pl.ds(row0, ROWS_PER_TILE)],
                    out_hbm.at[cid, pl.ds(row0, ROWS_PER_TILE)])

  return k(src3, dst3, g, zero_init)


def _sc_degree(dst3, zero_init):
  mesh = plsc.VectorSubcoreMesh(core_axis_name="c", subcore_axis_name="s", num_cores=NC, num_subcores=NS)

  @functools.partial(
      pl.kernel,
      out_type=jax.ShapeDtypeStruct((NC, N_ACC), jnp.float32),
      mesh=mesh,
      scratch_types=[
          pltpu.VMEM((CH, CB), jnp.int32),
          pltpu.VMEM((CB,), jnp.float32),
          pltpu.VMEM_SHARED((N_ACC,), jnp.float32),
      ],
  )
  def k(dst_hbm, zero_hbm, out_hbm, dstv, ones_v, acc):
    cid = lax.axis_index("c")
    sid = lax.axis_index("s")
    wid = sid * NC + cid
    pltpu.sync_copy(dst_hbm.at[wid], dstv)
    for i in range(CB // 16):
      ones_v[pl.ds(i * 16, 16)] = jnp.ones((16,), jnp.float32)
    row0 = sid * ROWS_PER_TILE
    pltpu.sync_copy(zero_hbm.at[pl.ds(row0, ROWS_PER_TILE)],
                    acc.at[pl.ds(row0, ROWS_PER_TILE)])
    plsc.subcore_barrier()

    def body(j, carry):
      pltpu.sync_copy(ones_v, acc.at[dstv.at[j]], add=True)
      return carry

    lax.fori_loop(0, CH, body, 0)
    plsc.subcore_barrier()
    pltpu.sync_copy(acc.at[pl.ds(row0, ROWS_PER_TILE)],
                    out_hbm.at[cid, pl.ds(row0, ROWS_PER_TILE)])

  return k(dst3, zero_init)


def _tc_prep(degp, xp, W1):
  def k(degp_ref, x_ref, w_ref, dinv_ref, g_ref):
    deg = degp_ref[0] + degp_ref[1] + 1.0
    dinv = lax.rsqrt(deg)
    dinv_ref[...] = dinv
    h = jnp.dot(x_ref[...], w_ref[...], preferred_element_type=jnp.float32)
    g_ref[...] = h * dinv

  return pl.pallas_call(
      k,
      out_shape=(jax.ShapeDtypeStruct((N_ACC, 1), jnp.float32),
                 jax.ShapeDtypeStruct((N_ACC, D), jnp.float32)),
  )(degp, xp, W1)


def _tc_layer(partials, g_prev, dinv, b_prev, W_next):
  dn = W_next.shape[1]

  def k(p_ref, g_ref, dinv_ref, b_ref, w_ref, x_ref, gn_ref):
    dinv = dinv_ref[...]
    xcur = jnp.tanh(dinv * (p_ref[0] + p_ref[1] + g_ref[...]) + b_ref[...])
    x_ref[...] = xcur
    h = jnp.dot(xcur, w_ref[...], preferred_element_type=jnp.float32)
    gn_ref[...] = h * dinv

  return pl.pallas_call(
      k,
      out_shape=(jax.ShapeDtypeStruct((N_ACC, D), jnp.float32),
                 jax.ShapeDtypeStruct((N_ACC, dn), jnp.float32)),
  )(partials, g_prev, dinv, b_prev, W_next)


def _tc_pool(p4, g4, dinv, b4, x1, x2, x3, batch_pad):
  def k(p4_ref, g4_ref, dinv_ref, b4_ref, x1_ref, x2_ref, x3_ref, batch_ref,
        o1_ref, o2_ref, o3_ref, o4_ref, K_ref, S_ref):
    x4 = jnp.tanh(dinv_ref[...] * (p4_ref[0] + p4_ref[1] + g4_ref[...])
                  + b4_ref[...])
    x4row = x4[0:NCOL, 0][None, :]

    rowid = lax.broadcasted_iota(jnp.int32, (16, NCOL), 0)
    col = lax.broadcasted_iota(jnp.int32, (16, NCOL), 1)
    in_graph = batch_ref[...] == rowid
    key = rowid.astype(jnp.float32) * jnp.float32(1e6) - x4row
    K_ref[...] = jnp.where(in_graph, key, INF)
    counts = jnp.sum(in_graph.astype(jnp.int32), axis=1, keepdims=True)
    S_ref[pl.ds(NUM_GRAPHS * SORTK, 6), :] = jnp.zeros((6, NCOL), jnp.float32)

    def step(t, carry):
      K = K_ref[...]
      m = jnp.min(K, axis=1, keepdims=True)
      hit = K == m
      a = jnp.min(jnp.where(hit, col, NCOL), axis=1, keepdims=True)
      sel = (col == a).astype(jnp.float32)
      valid = (t < counts).astype(jnp.float32)
      rows = sel * valid
      for g in range(NUM_GRAPHS):
        S_ref[pl.ds(g * SORTK + t, 1), :] = rows[g][None, :]
      K_ref[...] = jnp.where(col == a, INF, K)
      return carry

    lax.fori_loop(0, SORTK, step, 0)

    S = S_ref[...]
    o1_ref[...] = jnp.dot(S, x1_ref[0:NCOL, :], preferred_element_type=jnp.float32)
    o2_ref[...] = jnp.dot(S, x2_ref[0:NCOL, :], preferred_element_type=jnp.float32)
    o3_ref[...] = jnp.dot(S, x3_ref[0:NCOL, :], preferred_element_type=jnp.float32)
    o4_ref[...] = jnp.dot(S, x4[0:NCOL, :], preferred_element_type=jnp.float32)

  return pl.pallas_call(
      k,
      out_shape=(jax.ShapeDtypeStruct((336, D), jnp.float32),
                 jax.ShapeDtypeStruct((336, D), jnp.float32),
                 jax.ShapeDtypeStruct((336, D), jnp.float32),
                 jax.ShapeDtypeStruct((336, 1), jnp.float32)),
      scratch_shapes=[
          pltpu.VMEM((16, NCOL), jnp.float32),
          pltpu.VMEM((336, NCOL), jnp.float32),
      ],
  )(p4, g4, dinv, b4, x1, x2, x3, batch_pad)


def _tc_dense(P1, P2, P3, p4p, W5a, W5b, W5c, w5d, b5, W6r, b6, C3, cb1,
              cls2_W, cb2):
  def k(p1_ref, p2_ref, p3_ref, p4p_ref, w5a_ref, w5b_ref, w5c_ref, w5d_ref,
        b5_ref, w6_ref, b6_ref, c3_ref, cb1_ref, w2c_ref, cb2_ref, out_ref):
    P1, P2, P3, p4p = p1_ref[...], p2_ref[...], p3_ref[...], p4p_ref[...]
    o5 = (jnp.dot(P1, w5a_ref[...], preferred_element_type=jnp.float32)
          + jnp.dot(P2, w5b_ref[...], preferred_element_type=jnp.float32)
          + jnp.dot(P3, w5c_ref[...], preferred_element_type=jnp.float32)
          + p4p * w5d_ref[...] + b5_ref[...])
    o5 = jnp.maximum(o5, 0.0)

    ei = lax.broadcasted_iota(jnp.int32, (165, 336), 0)
    ej = lax.broadcasted_iota(jnp.int32, (165, 336), 1)
    gidx = ei // 15
    pidx = ei - gidx * 15
    sel_e = (ej == gidx * SORTK + 2 * pidx).astype(jnp.float32)
    sel_o = (ej == gidx * SORTK + 2 * pidx + 1).astype(jnp.float32)
    Me = jnp.dot(sel_e, o5, preferred_element_type=jnp.float32)
    Mo = jnp.dot(sel_o, o5, preferred_element_type=jnp.float32)
    M = jnp.maximum(Me, Mo)

    wi = lax.broadcasted_iota(jnp.int32, (121, 165), 0)
    wj = lax.broadcasted_iota(jnp.int32, (121, 165), 1)
    wg = wi // 11
    wp = wi - wg * 11
    wins = []
    for j in range(5):
      selj = (wj == wg * 15 + wp + j).astype(jnp.float32)
      wins.append(jnp.dot(selj, M, preferred_element_type=jnp.float32))
    win = jnp.concatenate(wins, axis=1)
    z = jnp.dot(win, w6_ref[...], preferred_element_type=jnp.float32)
    z = jnp.maximum(z + b6_ref[...], 0.0)

    h = jnp.sum(z[:, :, None] * c3_ref[...], axis=(0, 1), keepdims=True)
    h = jnp.maximum(h[0] + cb1_ref[...], 0.0)
    logits = jnp.dot(h, w2c_ref[...],
                     preferred_element_type=jnp.float32) + cb2_ref[...]
    mx = jnp.max(logits, axis=1, keepdims=True)
    ex = jnp.exp(logits - mx)
    out_ref[...] = ex / jnp.sum(ex, axis=1, keepdims=True)

  return pl.pallas_call(
      k,
      out_shape=jax.ShapeDtypeStruct((1, 10), jnp.float32),
  )(P1, P2, P3, p4p, W5a, W5b, W5c, w5d, b5, W6r, b6, C3, cb1, cls2_W, cb2)


def kernel(x, edge_index, gnn_batch, W1, b1, W2, b2, W3, b3, W4, b4,
           conv5_w, conv5_b, conv6_w, conv6_b, cls1_W, cls1_b, cls2_W, cls2_b):
  f32 = jnp.float32
  src, dst = edge_index[0], edge_index[1]
  pad_idx = (N + (jnp.arange(E_PAD - E, dtype=jnp.int32) % 128))
  src3 = jnp.concatenate([src, pad_idx]).reshape(NW, CH, CB)
  dst3 = jnp.concatenate([dst, pad_idx]).reshape(NW, CH, CB)

  xp = jnp.zeros((N_ACC, x.shape[1]), f32).at[:N].set(x)
  zeros_w = jnp.zeros((N_ACC, D), f32)
  zeros_s = jnp.zeros((N_ACC,), f32)
  batch_pad = jnp.full((1, NCOL), 15, jnp.int32).at[0, :N].set(gnn_batch)

  W5r = conv5_w.reshape(64, 3 * D + 1)
  W5a, W5b, W5c = W5r[:, 0:128].T, W5r[:, 128:256].T, W5r[:, 256:384].T
  w5d = W5r[:, 384][None, :]
  W6r = conv6_w.transpose(2, 1, 0).reshape(320, 128)
  C3 = cls1_W.reshape(11, 128, 11, 128).transpose(0, 2, 1, 3).reshape(121, 128, 128)

  degp = _sc_degree(dst3, zeros_s)
  dinv, g1 = _tc_prep(degp.reshape(NC, N_ACC, 1), xp, W1)

  p1 = _sc_edge_pass(src3, dst3, g1, zeros_w, wide=True)
  x1, g2 = _tc_layer(p1, g1, dinv, b1[None, :], W2)
  p2 = _sc_edge_pass(src3, dst3, g2, zeros_w, wide=True)
  x2, g3 = _tc_layer(p2, g2, dinv, b2[None, :], W3)
  p3 = _sc_edge_pass(src3, dst3, g3, zeros_w, wide=True)
  x3, g4 = _tc_layer(p3, g3, dinv, b3[None, :], W4)

  p4 = _sc_edge_pass(src3, dst3, g4[:, 0], zeros_s, wide=False)

  P1, P2, P3, p4p = _tc_pool(p4.reshape(NC, N_ACC, 1), g4, dinv, b4[None, :],
                             x1, x2, x3, batch_pad)
  probs = _tc_dense(P1, P2, P3, p4p, W5a, W5b, W5c, w5d, conv5_b[None, :],
                    W6r, conv6_b[None, :], C3, cls1_b[None, :], cls2_W,
                    cls2_b[None, :])
  return probs.reshape(10)
